# baseline (speedup 1.0000x reference)
import functools

import jax
import jax.numpy as jnp
from jax import lax
from jax.experimental import pallas as pl
from jax.experimental.pallas import tpu as pltpu
from jax.experimental.pallas import tpu_sc as plsc

VOCAB = 100000
DIM = 128
MAX_NORM = 1.0
B = 1024

TN = 2048
NT = pl.cdiv(VOCAB, TN)
LASTW = 1664
TAILC = VOCAB - (NT - 1) * TN - LASTW
TAILOFF = (NT - 1) * TN + LASTW
NBUF = 4
SPLIT = 2
CB = B // SPLIT

SC_CORES = 2
SC_SUBCORES = 16


def _make_sc_gather():
    nw = SC_CORES * SC_SUBCORES
    b_per_w = B // nw

    mesh = plsc.VectorSubcoreMesh(
        core_axis_name="c", subcore_axis_name="s", num_cores=SC_CORES
    )

    @functools.partial(
        pl.kernel,
        mesh=mesh,
        out_type=jax.ShapeDtypeStruct((B, DIM), jnp.float32),
        scratch_types=[
            pltpu.VMEM((b_per_w,), jnp.int32),
            pltpu.VMEM((b_per_w, DIM), jnp.float32),
            pltpu.SemaphoreType.DMA,
        ],
    )
    def gather(table_hbm, idx_hbm, out_hbm, idx_v, rows_v, sem):
        wid = lax.axis_index("s") * SC_CORES + lax.axis_index("c")
        base = wid * b_per_w
        pltpu.sync_copy(idx_hbm.at[pl.ds(base, b_per_w)], idx_v)
        pltpu.async_copy(table_hbm.at[idx_v], rows_v, sem).wait()
        pltpu.sync_copy(rows_v, out_hbm.at[pl.ds(base, b_per_w)])

    return gather


_get_sc_gather = functools.cache(_make_sc_gather)


def _main_copy(obuf, out_hbm, slot, s, col, width, sems):
    return pltpu.make_async_copy(
        obuf.at[slot, pl.ds(s * CB, CB), pl.ds(0, width)],
        out_hbm.at[pl.ds(s * CB, CB), pl.ds(col, width)],
        sems.at[slot, s],
    )


def _tail_copy(obuf, tail_hbm, slot, sems):
    return pltpu.make_async_copy(
        obuf.at[slot, :, pl.ds(1024, 1024)],
        tail_hbm,
        sems.at[slot, SPLIT],
    )


def _proj_body(x_ref, w_ref, b_ref, out_hbm, tail_hbm, xs_ref, obuf, sems):
    step = pl.program_id(0)
    slot = lax.rem(step, NBUF)

    @pl.when(step == 0)
    def _():
        x = x_ref[...]
        ss = jnp.sum(x * x, axis=1, keepdims=True)
        scale = jnp.minimum(1.0, MAX_NORM * lax.rsqrt(jnp.maximum(ss, 1e-14)))
        xs_ref[...] = (x * scale).astype(jnp.bfloat16)

    @pl.when(step >= NBUF)
    def _():
        col = pl.multiple_of((step - NBUF) * TN, TN)
        for s in range(SPLIT):
            _main_copy(obuf, out_hbm, slot, s, col, TN, sems).wait()

    w = w_ref[...].astype(jnp.bfloat16)
    acc = lax.dot_general(
        xs_ref[...], w, (((1,), (1,)), ((), ())),
        preferred_element_type=jnp.float32,
    )
    obuf[slot] = acc + b_ref[0]

    @pl.when(step < NT - 1)
    def _():
        col = pl.multiple_of(step * TN, TN)
        for s in range(SPLIT):
            _main_copy(obuf, out_hbm, slot, s, col, TN, sems).start()

    @pl.when(step == NT - 1)
    def _():
        for s in range(SPLIT):
            _main_copy(obuf, out_hbm, slot, s, (NT - 1) * TN, LASTW, sems).start()
        _tail_copy(obuf, tail_hbm, slot, sems).start()
        for k in range(NBUF):
            sk = NT - NBUF + k
            width = TN if sk < NT - 1 else LASTW
            for s in range(SPLIT):
                _main_copy(obuf, out_hbm, sk % NBUF, s, sk * TN, width, sems).wait()
        _tail_copy(obuf, tail_hbm, slot, sems).wait()


def _projection(x, w, b3d):
    return pl.pallas_call(
        _proj_body,
        grid=(NT,),
        in_specs=[
            pl.BlockSpec((B, DIM), lambda i: (0, 0)),
            pl.BlockSpec((TN, DIM), lambda i: (i, 0)),
            pl.BlockSpec((1, 1, TN), lambda i: (i, 0, 0)),
        ],
        out_specs=[
            pl.BlockSpec(memory_space=pltpu.MemorySpace.HBM),
            pl.BlockSpec(memory_space=pltpu.MemorySpace.HBM),
        ],
        out_shape=[
            jax.ShapeDtypeStruct((B, VOCAB), jnp.float32),
            jax.ShapeDtypeStruct((B, 1024), jnp.float32),
        ],
        scratch_shapes=[
            pltpu.VMEM((B, DIM), jnp.bfloat16),
            pltpu.VMEM((NBUF, B, TN), jnp.float32),
            pltpu.SemaphoreType.DMA((NBUF, SPLIT + 1)),
        ],
    )(x, w, b3d)


def _splice_body(_big_ref, tail_ref, o_ref):
    o_ref[...] = tail_ref[...]


def _splice(out, tail):
    return pl.pallas_call(
        _splice_body,
        grid=(1,),
        in_specs=[
            pl.BlockSpec((8, 128), lambda i: (0, 0)),
            pl.BlockSpec((B, 1024), lambda i: (0, 0)),
        ],
        out_specs=pl.BlockSpec((B, 1024), lambda i: (0, (VOCAB // 1024))),
        out_shape=jax.ShapeDtypeStruct((B, VOCAB), jnp.float32),
        input_output_aliases={0: 0},
    )(out, tail)


def kernel(_input, table, W, b):
    idx = _input.astype(jnp.int32)
    x = _get_sc_gather()(table, idx)
    b3d = jnp.pad(b, (0, NT * TN - VOCAB)).reshape(NT, 1, TN)
    out, tail = _projection(x, W, b3d)
    return _splice(out, tail)


# speedup vs baseline: 1.0090x; 1.0090x over previous
import functools

import jax
import jax.numpy as jnp
from jax import lax
from jax.experimental import pallas as pl
from jax.experimental.pallas import tpu as pltpu
from jax.experimental.pallas import tpu_sc as plsc

VOCAB = 100000
DIM = 128
MAX_NORM = 1.0
B = 1024

TB = 128
NB = B // TB
TN = 12544
NV = 8
LASTW = 12160
NBUF = 4

SC_CORES = 2
SC_SUBCORES = 16


def _make_sc_gather():
    nw = SC_CORES * SC_SUBCORES
    b_per_w = B // nw

    mesh = plsc.VectorSubcoreMesh(
        core_axis_name="c", subcore_axis_name="s", num_cores=SC_CORES
    )

    @functools.partial(
        pl.kernel,
        mesh=mesh,
        out_type=jax.ShapeDtypeStruct((B, DIM), jnp.float32),
        scratch_types=[
            pltpu.VMEM((b_per_w,), jnp.int32),
            pltpu.VMEM((b_per_w, DIM), jnp.float32),
            pltpu.SemaphoreType.DMA,
        ],
    )
    def gather(table_hbm, idx_hbm, out_hbm, idx_v, rows_v, sem):
        wid = lax.axis_index("s") * SC_CORES + lax.axis_index("c")
        base = wid * b_per_w
        pltpu.sync_copy(idx_hbm.at[pl.ds(base, b_per_w)], idx_v)
        pltpu.async_copy(table_hbm.at[idx_v], rows_v, sem).wait()
        pltpu.sync_copy(rows_v, out_hbm.at[pl.ds(base, b_per_w)])

    return gather


_get_sc_gather = functools.cache(_make_sc_gather)


def _main_copy(obuf, out_hbm, slot, row, col, width, sems):
    return pltpu.make_async_copy(
        obuf.at[slot, :, pl.ds(0, width)],
        out_hbm.at[pl.ds(row, TB), pl.ds(col, width)],
        sems.at[slot, 0],
    )


def _tail_copy(obuf, tail_hbm, slot, row, sems):
    return pltpu.make_async_copy(
        obuf.at[slot, :, pl.ds(LASTW, 128)],
        tail_hbm.at[pl.ds(row, TB), :],
        sems.at[slot, 1],
    )


def _proj_body(x_ref, w_ref, b_ref, out_hbm, tail_hbm, xs_ref, obuf, sems):
    step = pl.program_id(0)
    slot = lax.rem(step, NBUF)
    bt = lax.rem(step, NB)
    row = pl.multiple_of(bt * TB, TB)

    @pl.when(step < NB)
    def _():
        x = x_ref[...]
        ss = jnp.sum(x * x, axis=1, keepdims=True)
        scale = jnp.minimum(1.0, MAX_NORM * lax.rsqrt(jnp.maximum(ss, 1e-14)))
        xs_ref[pl.ds(row, TB), :] = (x * scale).astype(jnp.bfloat16)

    @pl.when((step >= NBUF) & (step < (NV - 1) * NB + NBUF))
    def _():
        ps = step - NBUF
        pcol = pl.multiple_of(lax.div(ps, NB) * TN, TN)
        prow = pl.multiple_of(lax.rem(ps, NB) * TB, TB)
        _main_copy(obuf, out_hbm, slot, prow, pcol, TN, sems).wait()

    @pl.when(step >= (NV - 1) * NB + NBUF)
    def _():
        ps = step - NBUF
        prow = pl.multiple_of(lax.rem(ps, NB) * TB, TB)
        _main_copy(obuf, out_hbm, slot, prow, (NV - 1) * TN, LASTW, sems).wait()
        _tail_copy(obuf, tail_hbm, slot, prow, sems).wait()

    w = w_ref[...].astype(jnp.bfloat16)
    acc = lax.dot_general(
        xs_ref[pl.ds(row, TB), :], w, (((1,), (1,)), ((), ())),
        preferred_element_type=jnp.float32,
    )
    obuf[slot] = acc + b_ref[0]

    @pl.when(step < (NV - 1) * NB)
    def _():
        col = pl.multiple_of(lax.div(step, NB) * TN, TN)
        _main_copy(obuf, out_hbm, slot, row, col, TN, sems).start()

    @pl.when(step >= (NV - 1) * NB)
    def _():
        _main_copy(obuf, out_hbm, slot, row, (NV - 1) * TN, LASTW, sems).start()
        _tail_copy(obuf, tail_hbm, slot, row, sems).start()

    @pl.when(step == NV * NB - 1)
    def _():
        for k in range(NBUF):
            sk = NV * NB - NBUF + k
            srow = (sk % NB) * TB
            _main_copy(obuf, out_hbm, sk % NBUF, srow, (NV - 1) * TN, LASTW, sems).wait()
            _tail_copy(obuf, tail_hbm, sk % NBUF, srow, sems).wait()


def _projection(x, w, b3d):
    assert NBUF <= NB
    return pl.pallas_call(
        _proj_body,
        grid=(NV * NB,),
        in_specs=[
            pl.BlockSpec((TB, DIM), lambda i: (i % NB, 0)),
            pl.BlockSpec((TN, DIM), lambda i: (i // NB, 0)),
            pl.BlockSpec((1, 1, TN), lambda i: (i // NB, 0, 0)),
        ],
        out_specs=[
            pl.BlockSpec(memory_space=pltpu.MemorySpace.HBM),
            pl.BlockSpec(memory_space=pltpu.MemorySpace.HBM),
        ],
        out_shape=[
            jax.ShapeDtypeStruct((B, VOCAB), jnp.float32),
            jax.ShapeDtypeStruct((B, 128), jnp.float32),
        ],
        scratch_shapes=[
            pltpu.VMEM((B, DIM), jnp.bfloat16),
            pltpu.VMEM((NBUF, TB, TN), jnp.float32),
            pltpu.SemaphoreType.DMA((NBUF, 2)),
        ],
    )(x, w, b3d)


def _splice_body(_big_ref, tail_ref, o_ref):
    o_ref[...] = tail_ref[...]


def _splice(out, tail):
    return pl.pallas_call(
        _splice_body,
        grid=(1,),
        in_specs=[
            pl.BlockSpec((8, 128), lambda i: (0, 0)),
            pl.BlockSpec((B, 128), lambda i: (0, 0)),
        ],
        out_specs=pl.BlockSpec((B, 128), lambda i: (0, VOCAB // 128)),
        out_shape=jax.ShapeDtypeStruct((B, VOCAB), jnp.float32),
        input_output_aliases={0: 0},
    )(out, tail)


def kernel(_input, table, W, b):
    idx = _input.astype(jnp.int32)
    x = _get_sc_gather()(table, idx)
    b3d = jnp.pad(b, (0, NV * TN - VOCAB)).reshape(NV, 1, TN)
    out, tail = _projection(x, W, b3d)
    return _splice(out, tail)
